# Initial kernel scaffold; baseline (speedup 1.0000x reference)
import jax
import jax.numpy as jnp
from jax.experimental import pallas as pl


def kernel(user_emb, item_emb, W_0_0, b_0_0, W_0_1, b_0_1, W_1_0, b_1_0, W_1_1, b_1_1, batch_data, edge_index_0, edge_index_1):
    raise NotImplementedError("write your pallas kernel here")


# speedup vs baseline: 15.6421x; 15.6421x over previous
import functools

import jax
import jax.numpy as jnp
from jax import lax
from jax.experimental import pallas as pl
from jax.experimental.pallas import tpu as pltpu
from jax.experimental.pallas import tpu_sc as plsc

NU = 20000
NI = 30000
N = NU + NI
NP = 50176
D = 64
HD = D // 2
HQ = D // 4
E = 800000
B = 1024
TEMP = 0.1
REG = 0.001

NC, NS = 2, 16
TILE_E = E // NS
ECH = 2000
NCH = TILE_E // ECH
ROWS_T = NP // NS
DEG_T = 3200
DEGP = DEG_T * NS
GR = 3 * B
GT = GR // NS

_mesh = plsc.VectorSubcoreMesh(core_axis_name="c", subcore_axis_name="s")
_sc_params = pltpu.CompilerParams(use_tc_tiling_on_sc=False)


@functools.partial(
    pl.kernel,
    out_type=jax.ShapeDtypeStruct((2 * DEGP,), jnp.float32),
    mesh=_mesh,
    compiler_params=_sc_params,
    scratch_types=[
        pltpu.VMEM((ECH,), jnp.int32),
        pltpu.VMEM((ECH,), jnp.float32),
        pltpu.VMEM((DEG_T,), jnp.float32),
        pltpu.VMEM_SHARED((DEGP,), jnp.float32),
    ],
)
def _deg_kernel(dstcat, zdeg, deg_out, idx_v, ones_v, stage_v, deg_s):
    c = lax.axis_index("c")
    s = lax.axis_index("s")

    def fill_ones(i, _):
        ones_v[pl.ds(i * 16, 16)] = jnp.full((16,), 1.0, jnp.float32)
        return 0

    lax.fori_loop(0, ECH // 16, fill_ones, 0)
    pltpu.sync_copy(zdeg, stage_v)
    pltpu.sync_copy(stage_v, deg_s.at[pl.ds(s * DEG_T, DEG_T)])
    plsc.subcore_barrier()

    def chunk(g, _):
        base = c * E + s * TILE_E + g * ECH
        pltpu.sync_copy(dstcat.at[pl.ds(base, ECH)], idx_v)
        pltpu.sync_copy(ones_v, deg_s.at[idx_v], add=True)
        return 0

    lax.fori_loop(0, NCH, chunk, 0)
    plsc.subcore_barrier()
    pltpu.sync_copy(deg_s.at[pl.ds(s * DEG_T, DEG_T)], stage_v)
    pltpu.sync_copy(stage_v, deg_out.at[pl.ds(c * DEGP + s * DEG_T, DEG_T)])


@functools.partial(
    pl.kernel,
    out_type=jax.ShapeDtypeStruct((4 * NP, HQ), jnp.float32),
    mesh=_mesh,
    compiler_params=_sc_params,
    scratch_types=[
        pltpu.VMEM((ECH,), jnp.int32),
        pltpu.VMEM((ECH,), jnp.int32),
        pltpu.VMEM((ECH, HQ), jnp.float32),
        pltpu.VMEM_SHARED((NP, HQ), jnp.float32),
        pltpu.SemaphoreType.DMA,
    ],
)
def _conv_kernel(y4, src4, dst, zacc, acc_out, sidx_v, didx_v, rows_v,
                 acc_s, sem):
    c = lax.axis_index("c")
    s = lax.axis_index("s")
    ztail = ROWS_T % ECH
    for p in range(2):
        q = p * 2 + c
        pltpu.sync_copy(zacc, rows_v)
        for k in range(ROWS_T // ECH):
            pltpu.sync_copy(rows_v,
                            acc_s.at[pl.ds(s * ROWS_T + k * ECH, ECH)])
        pltpu.sync_copy(rows_v.at[pl.ds(0, ztail)],
                        acc_s.at[pl.ds(s * ROWS_T + ROWS_T - ztail, ztail)])
        plsc.subcore_barrier()

        def chunk(g, _):
            base = s * TILE_E + g * ECH
            pltpu.sync_copy(src4.at[pl.ds(q * E + base, ECH)], sidx_v)
            pltpu.sync_copy(dst.at[pl.ds(base, ECH)], didx_v)
            pltpu.async_copy(y4.at[sidx_v], rows_v, sem).wait()
            pltpu.sync_copy(rows_v, acc_s.at[didx_v], add=True)
            return 0

        lax.fori_loop(0, NCH, chunk, 0)
        plsc.subcore_barrier()
        for k in range(ROWS_T // ECH):
            pltpu.sync_copy(acc_s.at[pl.ds(s * ROWS_T + k * ECH, ECH)],
                            rows_v)
            pltpu.sync_copy(
                rows_v,
                acc_out.at[pl.ds(q * NP + s * ROWS_T + k * ECH, ECH)])
        base = s * ROWS_T + ROWS_T - ztail
        pltpu.sync_copy(acc_s.at[pl.ds(base, ztail)],
                        rows_v.at[pl.ds(0, ztail)])
        pltpu.sync_copy(rows_v.at[pl.ds(0, ztail)],
                        acc_out.at[pl.ds(q * NP + base - s * ROWS_T
                                         + s * ROWS_T, ztail)])


@functools.partial(
    pl.kernel,
    out_type=jax.ShapeDtypeStruct((2, GR, D), jnp.float32),
    mesh=_mesh,
    compiler_params=_sc_params,
    scratch_types=[
        pltpu.VMEM((GT,), jnp.int32),
        pltpu.VMEM((GT, D), jnp.float32),
        pltpu.SemaphoreType.DMA,
    ],
)
def _gather_kernel(t0, t1, idx0, idx1, out, idx_v, rows_v, sem):
    c = lax.axis_index("c")
    s = lax.axis_index("s")

    @pl.when(c == 0)
    def _():
        pltpu.sync_copy(idx0.at[pl.ds(s * GT, GT)], idx_v)
        pltpu.async_copy(t0.at[idx_v], rows_v, sem).wait()
        pltpu.sync_copy(rows_v, out.at[0].at[pl.ds(s * GT, GT)])

    @pl.when(c == 1)
    def _():
        pltpu.sync_copy(idx1.at[pl.ds(s * GT, GT)], idx_v)
        pltpu.async_copy(t1.at[idx_v], rows_v, sem).wait()
        pltpu.sync_copy(rows_v, out.at[1].at[pl.ds(s * GT, GT)])


_RB = 1024
_GRID = NP // _RB


def _dis(deg_blk):
    return jnp.where(deg_blk > 0.0, lax.rsqrt(deg_blk), 0.0)


def _mm1_body(h_ref, w_ref, deg_ref, out_ref):
    dis = _dis(deg_ref[...])
    y = jnp.dot(h_ref[...], w_ref[...], preferred_element_type=jnp.float32)
    y = y * dis
    for qq in range(4):
        out_ref[qq] = y[:, qq * HQ:(qq + 1) * HQ]


def _mm1(h, w, deg):
    return pl.pallas_call(
        _mm1_body,
        grid=(_GRID,),
        in_specs=[
            pl.BlockSpec((_RB, D), lambda i: (i, 0)),
            pl.BlockSpec((D, D), lambda i: (0, 0)),
            pl.BlockSpec((_RB, 1), lambda i: (i, 0)),
        ],
        out_specs=pl.BlockSpec((4, _RB, HQ), lambda i: (0, i, 0)),
        out_shape=jax.ShapeDtypeStruct((4, NP, HQ), jnp.float32),
    )(h, w, deg)


def _mm2_body(acc_ref, deg_ref, b_ref, w_ref, out_ref):
    dis = _dis(deg_ref[...])
    h = jnp.concatenate([acc_ref[qq] for qq in range(4)],
                    axis=-1) * dis + b_ref[...]
    y = jnp.dot(h, w_ref[...], preferred_element_type=jnp.float32) * dis
    for qq in range(4):
        out_ref[qq] = y[:, qq * HQ:(qq + 1) * HQ]


def _mm2(acc, deg, b, w):
    return pl.pallas_call(
        _mm2_body,
        grid=(_GRID,),
        in_specs=[
            pl.BlockSpec((4, _RB, HQ), lambda i: (0, i, 0)),
            pl.BlockSpec((_RB, 1), lambda i: (i, 0)),
            pl.BlockSpec((1, D), lambda i: (0, 0)),
            pl.BlockSpec((D, D), lambda i: (0, 0)),
        ],
        out_specs=pl.BlockSpec((4, _RB, HQ), lambda i: (0, i, 0)),
        out_shape=jax.ShapeDtypeStruct((4, NP, HQ), jnp.float32),
    )(acc, deg, b, w)


def _post_body(acc_ref, deg_ref, b_ref, tot_ref, out_ref):
    dis = _dis(deg_ref[...])
    h = jnp.concatenate([acc_ref[qq] for qq in range(4)],
                    axis=-1) * dis + b_ref[...]
    nrm = jnp.sqrt(jnp.sum(h * h, axis=-1, keepdims=True))
    n = h / jnp.maximum(nrm, 1e-12)
    out_ref[...] = n + tot_ref[...]


def _post(acc, deg, b, tot):
    return pl.pallas_call(
        _post_body,
        grid=(_GRID,),
        in_specs=[
            pl.BlockSpec((4, _RB, HQ), lambda i: (0, i, 0)),
            pl.BlockSpec((_RB, 1), lambda i: (i, 0)),
            pl.BlockSpec((1, D), lambda i: (0, 0)),
            pl.BlockSpec((_RB, D), lambda i: (i, 0)),
        ],
        out_specs=pl.BlockSpec((_RB, D), lambda i: (i, 0)),
        out_shape=jax.ShapeDtypeStruct((NP, D), jnp.float32),
    )(acc, deg, b, tot)


def _cos(a, b):
    num = jnp.sum(a * b, axis=-1)
    den = jnp.maximum(
        jnp.sqrt(jnp.sum(a * a, axis=-1)) * jnp.sqrt(jnp.sum(b * b, axis=-1)),
        1e-8)
    return num / den


def _loss_body(g_ref, u_ref, i_ref, out_ref):
    loss = jnp.float32(0.0)
    for bi in range(2):
        cu = g_ref[bi, 0:B]
        pi = g_ref[bi, B:2 * B]
        ni = g_ref[bi, 2 * B:3 * B]
        s0 = jnp.sum(cu * pi, axis=-1)
        s1 = jnp.sum(cu * ni, axis=-1)
        bpr = -jnp.mean(jax.nn.log_sigmoid(s0 - s1))
        l0 = _cos(cu, pi) / TEMP
        l1 = _cos(cu, ni) / TEMP
        m = jnp.maximum(l0, l1)
        lse = m + jnp.log(jnp.exp(l0 - m) + jnp.exp(l1 - m))
        cl = jnp.mean(lse - l0)
        loss = loss + bpr + cl
    su = jnp.sqrt(jnp.sum(u_ref[...] * u_ref[...]))
    si = jnp.sqrt(jnp.sum(i_ref[...] * i_ref[...]))
    loss = loss + REG * (su + si) / NI
    out_ref[...] = jnp.reshape(loss, (1, 1))


def _loss(g, u_emb, i_emb):
    return pl.pallas_call(
        _loss_body,
        out_shape=jax.ShapeDtypeStruct((1, 1), jnp.float32),
    )(g, u_emb, i_emb)


def kernel(user_emb, item_emb, W_0_0, b_0_0, W_0_1, b_0_1,
           W_1_0, b_1_0, W_1_1, b_1_1, batch_data, edge_index_0,
           edge_index_1):
    params = [[(W_0_0, b_0_0.reshape(1, D)), (W_0_1, b_0_1.reshape(1, D))],
              [(W_1_0, b_1_0.reshape(1, D)), (W_1_1, b_1_1.reshape(1, D))]]
    edges = [edge_index_0, edge_index_1]

    total = jnp.concatenate([user_emb, item_emb], axis=0)
    total = jnp.pad(total, ((0, NP - N), (0, 0)))

    srcs, dsts = [], []
    for bi in range(2):
        src = edges[bi][0]
        srcs.append(jnp.concatenate(
            [src + qq * NP for qq in range(4)]))
        dsts.append(edges[bi][1])

    dstcat = jnp.concatenate(dsts)
    zdeg = jnp.zeros((DEG_T,), jnp.float32)
    zacc = jnp.zeros((ECH, HQ), jnp.float32)

    deg_flat = _deg_kernel(dstcat, zdeg)
    degs = [deg_flat[:NP].reshape(NP, 1),
            deg_flat[DEGP:DEGP + NP].reshape(NP, 1)]

    all_emb = []
    for bi in range(2):
        (w0, bb0), (w1, bb1) = params[bi]
        y = _mm1(total, w0, degs[bi])
        acc = _conv_kernel(y.reshape(4 * NP, HQ), srcs[bi], dsts[bi],
                           zacc).reshape(4, NP, HQ)
        y = _mm2(acc, degs[bi], bb0, w1)
        acc = _conv_kernel(y.reshape(4 * NP, HQ), srcs[bi], dsts[bi],
                           zacc).reshape(4, NP, HQ)
        total = _post(acc, degs[bi], bb1, total)
        all_emb.append(total)

    idxs = []
    for bi in range(2):
        data = batch_data[:, bi]
        idxs.append(jnp.concatenate(
            [data[:, 0], NU + data[:, 1], NU + data[:, 2]]).astype(jnp.int32))

    g = _gather_kernel(all_emb[0], all_emb[1], idxs[0], idxs[1])
    return _loss(g, user_emb, item_emb)[0, 0]
